# Initial kernel scaffold; baseline (speedup 1.0000x reference)
import jax
import jax.numpy as jnp
from jax.experimental import pallas as pl


def kernel(x, W_emb, b_emb, pos_emb_enc, W_enc, b_enc, mask_token, pos_emb_pred, W_pred, b_pred, mask_noise):
    raise NotImplementedError("write your pallas kernel here")


# speedup vs baseline: 11.0299x; 11.0299x over previous
import functools

import jax
import jax.numpy as jnp
from jax.experimental import pallas as pl

B, P, DIN, D = 16, 4096, 128, 128
N_MASK = P // 2
LN_EPS = 1e-5


def _sel_prologue_kernel(noise_ref, pos_enc_ref, pos_pred_ref, w_emb_ref,
                         w_enc_ref, w_pred_ref, b_emb_ref, b_enc_ref,
                         mtok_ref, b_pred_ref, mask_ref, maskt_ref, pe2_ref,
                         pp2_ref, wc_ref, mv_ref):
    noise = noise_ref[...]
    bits = jax.lax.bitcast_convert_type(noise, jnp.int32)
    idxrev = (P - 1) - jax.lax.broadcasted_iota(jnp.int32, (B, P), 1)

    active = jnp.ones((B, P), dtype=jnp.bool_)
    k = jnp.full((B, 1), N_MASK, dtype=jnp.int32)
    tv = jnp.zeros((B, 1), dtype=jnp.int32)
    tir = jnp.zeros((B, 1), dtype=jnp.int32)

    passes = [(0, 24), (0, 18), (0, 12), (0, 6), (0, 0), (1, 6), (1, 0)]
    for src, shift in passes:
        dsrc = bits if src == 0 else idxrev
        d = jax.lax.shift_right_logical(dsrc, shift) & 63
        s = jnp.zeros((B, 1), dtype=jnp.int32)
        for m in (32, 16, 8, 4, 2, 1):
            cand = s + m
            cnt = jnp.sum(
                jnp.where(active & (d >= cand), 1, 0).astype(jnp.int32),
                axis=1, keepdims=True)
            s = jnp.where(cnt >= k, cand, s)
        cnt_gt = jnp.sum(
            jnp.where(active & (d > s), 1, 0).astype(jnp.int32),
            axis=1, keepdims=True)
        k = k - cnt_gt
        active = active & (d == s)
        if src == 0:
            tv = tv | jax.lax.shift_left(s, shift)
        else:
            tir = tir | jax.lax.shift_left(s, shift)

    masked = (bits > tv) | ((bits == tv) & (idxrev >= tir))
    mask_i = masked.astype(jnp.int32)
    mask_ref[...] = mask_i
    maskt_ref[...] = mask_i.T

    f32 = jnp.float32
    pe2_ref[...] = (
        jnp.dot(pos_enc_ref[...] + b_emb_ref[...], w_enc_ref[...],
                preferred_element_type=f32) + b_enc_ref[...])
    pp2_ref[...] = jnp.dot(pos_pred_ref[...], w_pred_ref[...],
                           preferred_element_type=f32)
    wc_ref[...] = jnp.dot(w_emb_ref[...], w_enc_ref[...],
                          preferred_element_type=f32)
    mv_ref[...] = (jnp.dot(mtok_ref[...], w_pred_ref[...],
                           preferred_element_type=f32) + b_pred_ref[...])


def _main_kernel(x_ref, mt_ref, pe2_ref, pp2_ref, wc_ref, wp_ref, mv_ref,
                 out_ref):
    xb = x_ref[0]
    c = jnp.dot(xb, wc_ref[...], preferred_element_type=jnp.float32)
    c = c + pe2_ref[...]
    mu = jnp.mean(c, axis=1, keepdims=True)
    cm = c - mu
    var = jnp.mean(cm * cm, axis=1, keepdims=True)
    ln = cm * jax.lax.rsqrt(var + LN_EPS)
    pid = pl.program_id(0)
    laneiota = jax.lax.broadcasted_iota(jnp.int32, (P, B), 1)
    mcol = jnp.sum(jnp.where(laneiota == pid, mt_ref[...], 0),
                   axis=1, keepdims=True)
    validc = mcol == 0
    s = jnp.sum(jnp.where(validc, ln, 0.0), axis=0, keepdims=True)
    ctx_mean = s * (1.0 / (P - N_MASK))
    v = jnp.dot(ctx_mean, wp_ref[...],
                preferred_element_type=jnp.float32) + mv_ref[...]
    out_ref[0] = jnp.where(validc, c, pp2_ref[...] + v)


def kernel(x, W_emb, b_emb, pos_emb_enc, W_enc, b_enc, mask_token,
           pos_emb_pred, W_pred, b_pred, mask_noise):
    f32 = jnp.float32
    b_emb2 = b_emb.reshape(1, D)
    b_enc2 = b_enc.reshape(1, D)
    mtok2 = mask_token.reshape(1, D)
    b_pred2 = b_pred.reshape(1, D)

    maskI, maskT, pe2, pp2, wc, mv = pl.pallas_call(
        _sel_prologue_kernel,
        out_shape=(
            jax.ShapeDtypeStruct((B, P), jnp.int32),
            jax.ShapeDtypeStruct((P, B), jnp.int32),
            jax.ShapeDtypeStruct((P, D), f32),
            jax.ShapeDtypeStruct((P, D), f32),
            jax.ShapeDtypeStruct((DIN, D), f32),
            jax.ShapeDtypeStruct((1, D), f32),
        ),
    )(mask_noise, pos_emb_enc, pos_emb_pred, W_emb, W_enc, W_pred,
      b_emb2, b_enc2, mtok2, b_pred2)

    x_like = pl.pallas_call(
        _main_kernel,
        grid=(B,),
        in_specs=[
            pl.BlockSpec((1, P, DIN), lambda b: (b, 0, 0)),
            pl.BlockSpec((P, B), lambda b: (0, 0)),
            pl.BlockSpec((P, D), lambda b: (0, 0)),
            pl.BlockSpec((P, D), lambda b: (0, 0)),
            pl.BlockSpec((DIN, D), lambda b: (0, 0)),
            pl.BlockSpec((D, D), lambda b: (0, 0)),
            pl.BlockSpec((1, D), lambda b: (0, 0)),
        ],
        out_specs=pl.BlockSpec((1, P, D), lambda b: (b, 0, 0)),
        out_shape=jax.ShapeDtypeStruct((B, P, D), f32),
    )(x, maskT, pe2, pp2, wc, W_pred, mv)

    return x_like, maskI.astype(jnp.bool_)


# speedup vs baseline: 12.9160x; 1.1710x over previous
import functools

import jax
import jax.numpy as jnp
from jax.experimental import pallas as pl

B, P, DIN, D = 16, 4096, 128, 128
N_MASK = P // 2
LN_EPS = 1e-5


def _sel_prologue_kernel(noise_ref, pos_enc_ref, pos_pred_ref, w_emb_ref,
                         w_enc_ref, w_pred_ref, b_emb_ref, b_enc_ref,
                         mtok_ref, b_pred_ref, mask_ref, maskt_ref, pe2_ref,
                         pp2_ref, wc_ref, mv_ref):
    noise = noise_ref[...]
    bits = jax.lax.bitcast_convert_type(noise, jnp.int32)
    idxrev = (P - 1) - jax.lax.broadcasted_iota(jnp.int32, (B, P), 1)

    active = jnp.ones((B, P), dtype=jnp.bool_)
    k = jnp.full((B, 1), N_MASK, dtype=jnp.int32)
    tv = jnp.zeros((B, 1), dtype=jnp.int32)
    tir = jnp.zeros((B, 1), dtype=jnp.int32)

    passes = [(0, 24), (0, 18), (0, 12), (0, 6), (0, 0), (1, 6), (1, 0)]
    for src, shift in passes:
        dsrc = bits if src == 0 else idxrev
        d = jax.lax.shift_right_logical(dsrc, shift) & 63
        s = jnp.zeros((B, 1), dtype=jnp.int32)
        for m in (32, 16, 8, 4, 2, 1):
            cand = s + m
            cnt = jnp.sum(
                jnp.where(active & (d >= cand), 1, 0).astype(jnp.int32),
                axis=1, keepdims=True)
            s = jnp.where(cnt >= k, cand, s)
        cnt_gt = jnp.sum(
            jnp.where(active & (d > s), 1, 0).astype(jnp.int32),
            axis=1, keepdims=True)
        k = k - cnt_gt
        active = active & (d == s)
        if src == 0:
            tv = tv | jax.lax.shift_left(s, shift)
        else:
            tir = tir | jax.lax.shift_left(s, shift)

    masked = (bits > tv) | ((bits == tv) & (idxrev >= tir))
    mask_i = masked.astype(jnp.int32)
    mask_ref[...] = mask_i
    maskt_ref[...] = masked.astype(jnp.float32).T

    f32 = jnp.float32
    pe2_ref[...] = (
        jnp.dot(pos_enc_ref[...] + b_emb_ref[...], w_enc_ref[...],
                preferred_element_type=f32) + b_enc_ref[...])
    pp2_ref[...] = jnp.dot(pos_pred_ref[...], w_pred_ref[...],
                           preferred_element_type=f32)
    wc_ref[...] = jnp.dot(w_emb_ref[...], w_enc_ref[...],
                          preferred_element_type=f32)
    mv_ref[...] = (jnp.dot(mtok_ref[...], w_pred_ref[...],
                           preferred_element_type=f32) + b_pred_ref[...])


def _main_kernel(x_ref, mr_ref, mt_ref, pe2_ref, pp2_ref, wc_ref, wp_ref,
                 mv_ref, out_ref):
    f32 = jnp.float32
    xb = x_ref[0]
    c = jnp.dot(xb, wc_ref[...], preferred_element_type=f32)
    c = c + pe2_ref[...]
    o_mat = jnp.full((D, D), 1.0 / D, dtype=f32)
    mu = jnp.dot(c, o_mat, preferred_element_type=f32)
    cm = c - mu
    var = jnp.dot(cm * cm, o_mat, preferred_element_type=f32)
    ln = cm * jax.lax.rsqrt(var + LN_EPS)
    vrow = 1.0 - mr_ref[0].astype(f32)
    s = jnp.dot(vrow, ln, preferred_element_type=f32)
    ctx_mean = s * (1.0 / (P - N_MASK))
    v = jnp.dot(ctx_mean, wp_ref[...],
                preferred_element_type=f32) + mv_ref[...]
    pid = pl.program_id(0)
    subiota = jax.lax.broadcasted_iota(jnp.int32, (B, D), 0)
    onehot = jnp.where(subiota == pid, 1.0, 0.0).astype(f32)
    mfull = jnp.dot(mt_ref[...], onehot, preferred_element_type=f32)
    out_ref[0] = c + mfull * (pp2_ref[...] + v - c)


def kernel(x, W_emb, b_emb, pos_emb_enc, W_enc, b_enc, mask_token,
           pos_emb_pred, W_pred, b_pred, mask_noise):
    f32 = jnp.float32
    b_emb2 = b_emb.reshape(1, D)
    b_enc2 = b_enc.reshape(1, D)
    mtok2 = mask_token.reshape(1, D)
    b_pred2 = b_pred.reshape(1, D)

    maskI, maskT, pe2, pp2, wc, mv = pl.pallas_call(
        _sel_prologue_kernel,
        out_shape=(
            jax.ShapeDtypeStruct((B, P), jnp.int32),
            jax.ShapeDtypeStruct((P, B), f32),
            jax.ShapeDtypeStruct((P, D), f32),
            jax.ShapeDtypeStruct((P, D), f32),
            jax.ShapeDtypeStruct((DIN, D), f32),
            jax.ShapeDtypeStruct((1, D), f32),
        ),
    )(mask_noise, pos_emb_enc, pos_emb_pred, W_emb, W_enc, W_pred,
      b_emb2, b_enc2, mtok2, b_pred2)

    x_like = pl.pallas_call(
        _main_kernel,
        grid=(B,),
        in_specs=[
            pl.BlockSpec((1, P, DIN), lambda b: (b, 0, 0)),
            pl.BlockSpec((1, 1, P), lambda b: (b, 0, 0)),
            pl.BlockSpec((P, B), lambda b: (0, 0)),
            pl.BlockSpec((P, D), lambda b: (0, 0)),
            pl.BlockSpec((P, D), lambda b: (0, 0)),
            pl.BlockSpec((DIN, D), lambda b: (0, 0)),
            pl.BlockSpec((D, D), lambda b: (0, 0)),
            pl.BlockSpec((1, D), lambda b: (0, 0)),
        ],
        out_specs=pl.BlockSpec((1, P, D), lambda b: (b, 0, 0)),
        out_shape=jax.ShapeDtypeStruct((B, P, D), f32),
    )(x, maskI.reshape(B, 1, P), maskT, pe2, pp2, wc, W_pred, mv)

    return x_like, maskI.astype(jnp.bool_)


# speedup vs baseline: 14.7754x; 1.1440x over previous
import jax
import jax.numpy as jnp
from jax.experimental import pallas as pl
from jax.experimental.pallas import tpu as pltpu

B, P, DIN, D = 16, 4096, 128, 128
N_MASK = P // 2
LN_EPS = 1e-5


def _fused_kernel(x_ref, noise_ref, pos_enc_ref, pos_pred_ref, w_emb_ref,
                  w_enc_ref, w_pred_ref, b_emb_ref, b_enc_ref, mtok_ref,
                  b_pred_ref, out_ref, mask_ref, mrow_s, mt_s, pe2_s, pp2_s,
                  wc_s, mv_s):
    f32 = jnp.float32
    pid = pl.program_id(0)

    @pl.when(pid == 0)
    def _prologue():
        noise = noise_ref[...]
        bits = jax.lax.bitcast_convert_type(noise, jnp.int32)
        idxrev = (P - 1) - jax.lax.broadcasted_iota(jnp.int32, (B, P), 1)

        active = jnp.ones((B, P), dtype=jnp.bool_)
        k = jnp.full((B, 1), N_MASK, dtype=jnp.int32)
        tv = jnp.zeros((B, 1), dtype=jnp.int32)
        tir = jnp.zeros((B, 1), dtype=jnp.int32)

        passes = [(0, 24), (0, 18), (0, 12), (0, 6), (0, 0), (1, 6), (1, 0)]
        for src, shift in passes:
            dsrc = bits if src == 0 else idxrev
            d = jax.lax.shift_right_logical(dsrc, shift) & 63
            s = jnp.zeros((B, 1), dtype=jnp.int32)
            for m in (32, 16, 8, 4, 2, 1):
                cand = s + m
                cnt = jnp.sum(
                    jnp.where(active & (d >= cand), 1, 0).astype(jnp.int32),
                    axis=1, keepdims=True)
                s = jnp.where(cnt >= k, cand, s)
            cnt_gt = jnp.sum(
                jnp.where(active & (d > s), 1, 0).astype(jnp.int32),
                axis=1, keepdims=True)
            k = k - cnt_gt
            active = active & (d == s)
            if src == 0:
                tv = tv | jax.lax.shift_left(s, shift)
            else:
                tir = tir | jax.lax.shift_left(s, shift)

        masked = (bits > tv) | ((bits == tv) & (idxrev >= tir))
        mask_ref[...] = masked.astype(jnp.int32)
        maskf = masked.astype(f32)
        mrow_s[...] = maskf
        mt_s[...] = maskf.T

        pe2_s[...] = (
            jnp.dot(pos_enc_ref[...] + b_emb_ref[...], w_enc_ref[...],
                    preferred_element_type=f32) + b_enc_ref[...])
        pp2_s[...] = jnp.dot(pos_pred_ref[...], w_pred_ref[...],
                             preferred_element_type=f32)
        wc_s[...] = jnp.dot(w_emb_ref[...], w_enc_ref[...],
                            preferred_element_type=f32)
        mv_s[...] = (jnp.dot(mtok_ref[...], w_pred_ref[...],
                             preferred_element_type=f32) + b_pred_ref[...])

    f32 = jnp.float32
    xb = x_ref[0]
    pe2 = pe2_s[...]
    c = jnp.dot(xb, wc_s[...], preferred_element_type=f32) + pe2
    o_mat = jnp.full((D, D), 1.0 / D, dtype=f32)
    mu = jnp.dot(c, o_mat, preferred_element_type=f32)
    cm = c - mu
    var = jnp.dot(cm * cm, o_mat, preferred_element_type=f32)
    ln = cm * jax.lax.rsqrt(var + LN_EPS)
    vrow = 1.0 - mrow_s[pl.ds(pid, 1), :]
    s = jnp.dot(vrow, ln, preferred_element_type=f32)
    ctx_mean = s * (1.0 / (P - N_MASK))
    v = jnp.dot(ctx_mean, w_pred_ref[...],
                preferred_element_type=f32) + mv_s[...]
    subiota = jax.lax.broadcasted_iota(jnp.int32, (B, D), 0)
    onehot = jnp.where(subiota == pid, 1.0, 0.0).astype(f32)
    mfull = jnp.dot(mt_s[...], onehot, preferred_element_type=f32)
    out_ref[0] = c + mfull * (pp2_s[...] + v - c)


def kernel(x, W_emb, b_emb, pos_emb_enc, W_enc, b_enc, mask_token,
           pos_emb_pred, W_pred, b_pred, mask_noise):
    f32 = jnp.float32
    b_emb2 = b_emb.reshape(1, D)
    b_enc2 = b_enc.reshape(1, D)
    mtok2 = mask_token.reshape(1, D)
    b_pred2 = b_pred.reshape(1, D)

    const = lambda i: (0, 0)
    x_like, maskI = pl.pallas_call(
        _fused_kernel,
        grid=(B,),
        in_specs=[
            pl.BlockSpec((1, P, DIN), lambda i: (i, 0, 0)),
            pl.BlockSpec((B, P), const),
            pl.BlockSpec((P, D), const),
            pl.BlockSpec((P, D), const),
            pl.BlockSpec((DIN, D), const),
            pl.BlockSpec((D, D), const),
            pl.BlockSpec((D, D), const),
            pl.BlockSpec((1, D), const),
            pl.BlockSpec((1, D), const),
            pl.BlockSpec((1, D), const),
            pl.BlockSpec((1, D), const),
        ],
        out_specs=(
            pl.BlockSpec((1, P, D), lambda i: (i, 0, 0)),
            pl.BlockSpec((B, P), const),
        ),
        out_shape=(
            jax.ShapeDtypeStruct((B, P, D), f32),
            jax.ShapeDtypeStruct((B, P), jnp.int32),
        ),
        scratch_shapes=[
            pltpu.VMEM((B, P), f32),
            pltpu.VMEM((P, B), f32),
            pltpu.VMEM((P, D), f32),
            pltpu.VMEM((P, D), f32),
            pltpu.VMEM((DIN, D), f32),
            pltpu.VMEM((1, D), f32),
        ],
    )(x, mask_noise, pos_emb_enc, pos_emb_pred, W_emb, W_enc, W_pred,
      b_emb2, b_enc2, mtok2, b_pred2)

    return x_like, maskI.astype(jnp.bool_)


# speedup vs baseline: 14.7784x; 1.0002x over previous
import jax
import jax.numpy as jnp
from jax.experimental import pallas as pl
from jax.experimental.pallas import tpu as pltpu

B, P, DIN, D = 16, 4096, 128, 128
N_MASK = P // 2
LN_EPS = 1e-5


def _fused_kernel(x_ref, noise_ref, pos_enc_ref, pos_pred_ref, w_emb_ref,
                  w_enc_ref, w_pred_ref, b_emb_ref, b_enc_ref, mtok_ref,
                  b_pred_ref, out_ref, mask_ref, mrow_s, mt_s, pe2_s, pp2_s,
                  wc_s, mv_s):
    f32 = jnp.float32
    pid = pl.program_id(0)

    @pl.when(pid == 0)
    def _prologue():
        noise = noise_ref[...]
        bits = jax.lax.bitcast_convert_type(noise, jnp.int32)
        idxrev = (P - 1) - jax.lax.broadcasted_iota(jnp.int32, (B, P), 1)

        active = jnp.ones((B, P), dtype=jnp.bool_)
        k = jnp.full((B, 1), N_MASK, dtype=jnp.int32)
        tv = jnp.zeros((B, 1), dtype=jnp.int32)
        tir = jnp.zeros((B, 1), dtype=jnp.int32)

        passes = [(0, 24), (0, 18), (0, 12), (0, 6), (0, 0), (1, 6), (1, 0)]
        for src, shift in passes:
            dsrc = bits if src == 0 else idxrev
            d = jax.lax.shift_right_logical(dsrc, shift) & 63
            dm = jnp.where(active, d, -1)
            s = jnp.zeros((B, 1), dtype=jnp.int32)
            for m in (32, 16, 8, 4, 2, 1):
                cand = s + m
                cnt = jnp.sum((dm >= cand).astype(jnp.int32),
                              axis=1, keepdims=True)
                s = jnp.where(cnt >= k, cand, s)
            cnt_gt = jnp.sum((dm > s).astype(jnp.int32),
                             axis=1, keepdims=True)
            k = k - cnt_gt
            active = dm == s
            if src == 0:
                tv = tv | jax.lax.shift_left(s, shift)
            else:
                tir = tir | jax.lax.shift_left(s, shift)

        masked = (bits > tv) | ((bits == tv) & (idxrev >= tir))
        mask_ref[...] = masked.astype(jnp.int32)
        mrow_s[...] = masked.astype(f32)
        mt_s[...] = masked.astype(jnp.bfloat16).T

        pe2_s[...] = (
            jnp.dot(pos_enc_ref[...] + b_emb_ref[...], w_enc_ref[...],
                    preferred_element_type=f32) + b_enc_ref[...])
        pp2_s[...] = jnp.dot(pos_pred_ref[...], w_pred_ref[...],
                             preferred_element_type=f32)
        wc_s[...] = jnp.dot(w_emb_ref[...], w_enc_ref[...],
                            preferred_element_type=f32)
        mv_s[...] = (jnp.dot(mtok_ref[...], w_pred_ref[...],
                             preferred_element_type=f32) + b_pred_ref[...])

    f32 = jnp.float32
    bf16 = jnp.bfloat16
    xb = x_ref[0]
    pe2 = pe2_s[...]
    c = jnp.dot(xb, wc_s[...], preferred_element_type=f32) + pe2
    o_mat = jnp.full((D, D), 1.0 / D, dtype=bf16)
    mu = jnp.dot(c.astype(bf16), o_mat, preferred_element_type=f32)
    cm = c - mu
    var = jnp.dot((cm * cm).astype(bf16), o_mat, preferred_element_type=f32)
    ln = cm * jax.lax.rsqrt(var + LN_EPS)
    vrow = (1.0 - mrow_s[pl.ds(pid, 1), :]).astype(bf16)
    s = jnp.dot(vrow, ln.astype(bf16), preferred_element_type=f32)
    ctx_mean = s * (1.0 / (P - N_MASK))
    v = jnp.dot(ctx_mean, w_pred_ref[...],
                preferred_element_type=f32) + mv_s[...]
    subiota = jax.lax.broadcasted_iota(jnp.int32, (B, D), 0)
    onehot = jnp.where(subiota == pid, 1.0, 0.0).astype(bf16)
    mfull = jnp.dot(mt_s[...], onehot, preferred_element_type=f32)
    out_ref[0] = c + mfull * (pp2_s[...] + v - c)


def kernel(x, W_emb, b_emb, pos_emb_enc, W_enc, b_enc, mask_token,
           pos_emb_pred, W_pred, b_pred, mask_noise):
    f32 = jnp.float32
    b_emb2 = b_emb.reshape(1, D)
    b_enc2 = b_enc.reshape(1, D)
    mtok2 = mask_token.reshape(1, D)
    b_pred2 = b_pred.reshape(1, D)

    const = lambda i: (0, 0)
    x_like, maskI = pl.pallas_call(
        _fused_kernel,
        grid=(B,),
        in_specs=[
            pl.BlockSpec((1, P, DIN), lambda i: (i, 0, 0)),
            pl.BlockSpec((B, P), const),
            pl.BlockSpec((P, D), const),
            pl.BlockSpec((P, D), const),
            pl.BlockSpec((DIN, D), const),
            pl.BlockSpec((D, D), const),
            pl.BlockSpec((D, D), const),
            pl.BlockSpec((1, D), const),
            pl.BlockSpec((1, D), const),
            pl.BlockSpec((1, D), const),
            pl.BlockSpec((1, D), const),
        ],
        out_specs=(
            pl.BlockSpec((1, P, D), lambda i: (i, 0, 0)),
            pl.BlockSpec((B, P), const),
        ),
        out_shape=(
            jax.ShapeDtypeStruct((B, P, D), f32),
            jax.ShapeDtypeStruct((B, P), jnp.int32),
        ),
        scratch_shapes=[
            pltpu.VMEM((B, P), f32),
            pltpu.VMEM((P, B), jnp.bfloat16),
            pltpu.VMEM((P, D), f32),
            pltpu.VMEM((P, D), f32),
            pltpu.VMEM((DIN, D), f32),
            pltpu.VMEM((1, D), f32),
        ],
    )(x, mask_noise, pos_emb_enc, pos_emb_pred, W_emb, W_enc, W_pred,
      b_emb2, b_enc2, mtok2, b_pred2)

    return x_like, maskI.astype(jnp.bool_)
